# Initial kernel scaffold; baseline (speedup 1.0000x reference)
import jax
import jax.numpy as jnp
from jax.experimental import pallas as pl


def kernel(x, edge_index, W1_0, b1_0, W2_0, b2_0, eps0, gamma0, beta0, W1_1, b1_1, W2_1, b2_1, eps1, gamma1, beta1):
    raise NotImplementedError("write your pallas kernel here")


# speedup vs baseline: 7.5376x; 7.5376x over previous
import functools

import jax
import jax.numpy as jnp
from jax import lax
from jax.experimental import pallas as pl
from jax.experimental.pallas import tpu as pltpu
from jax.experimental.pallas import tpu_sc as plsc

_BN_EPS = 1e-5
_NC = 2
_NS = 16
_BATCH = 128
_ROWS = 1024
_ZR = 16


def _mm_body(x_ref, w_ref, o_ref):
    o_ref[...] = jnp.dot(x_ref[...], w_ref[...],
                         preferred_element_type=jnp.float32)


def _linear_tc(x, W):
    M, K = x.shape
    F = W.shape[1]
    grid = M // _ROWS
    return pl.pallas_call(
        _mm_body,
        grid=(grid,),
        in_specs=[
            pl.BlockSpec((_ROWS, K), lambda i: (i, 0)),
            pl.BlockSpec((K, F), lambda i: (0, 0)),
        ],
        out_specs=pl.BlockSpec((_ROWS, F), lambda i: (i, 0)),
        out_shape=jax.ShapeDtypeStruct((M, F), jnp.float32),
    )(x, W)


def _segment_sum_sc(y, src_w, dst_w, n_pad, feat, nb):
    rows_per_tile = n_pad // _NS

    def body(y_hbm, src_hbm, dst_hbm, out_hbm, src_v, dst_v, rows_v,
             zbuf_v, acc_sh, sem):
        c = lax.axis_index("c")
        s = lax.axis_index("s")
        wid = s * _NC + c
        for r in range(_ZR):
            for q in range(feat // 16):
                zbuf_v[r, pl.ds(q * 16, 16)] = jnp.zeros((16,), jnp.float32)
        base = s * rows_per_tile

        def zloop(i, carry):
            pltpu.sync_copy(zbuf_v, acc_sh.at[pl.ds(base + i * _ZR, _ZR)])
            return carry

        lax.fori_loop(0, rows_per_tile // _ZR, zloop, 0)
        pltpu.sync_copy(src_hbm.at[wid], src_v)
        pltpu.sync_copy(dst_hbm.at[wid], dst_v)
        plsc.subcore_barrier()

        def eloop(j, carry):
            pltpu.async_copy(y_hbm.at[src_v.at[j]], rows_v, sem).wait()
            pltpu.sync_copy(rows_v, acc_sh.at[dst_v.at[j]], add=True)
            return carry

        lax.fori_loop(0, nb, eloop, 0)
        plsc.subcore_barrier()
        pltpu.sync_copy(acc_sh.at[pl.ds(base, rows_per_tile)],
                        out_hbm.at[c, pl.ds(base, rows_per_tile)])

    k = pl.kernel(
        body,
        out_type=jax.ShapeDtypeStruct((_NC, n_pad, feat), jnp.float32),
        mesh=plsc.VectorSubcoreMesh(core_axis_name="c", subcore_axis_name="s"),
        scratch_types=[
            pltpu.VMEM((nb, _BATCH), jnp.int32),
            pltpu.VMEM((nb, _BATCH), jnp.int32),
            pltpu.VMEM((_BATCH, feat), jnp.float32),
            pltpu.VMEM((_ZR, feat), jnp.float32),
            pltpu.VMEM_SHARED((n_pad, feat), jnp.float32),
            pltpu.SemaphoreType.DMA,
        ],
        compiler_params=pltpu.CompilerParams(use_tc_tiling_on_sc=False),
    )
    return k(y, src_w, dst_w)


def _block_body(y_ref, a0_ref, a1_ref, em_ref, b1_ref, w2_ref, b2_ref,
                g_ref, be_ref, w1n_ref, o_ref):
    z = em_ref[...] * y_ref[...] + a0_ref[...] + a1_ref[...] + b1_ref[...]
    z = jnp.maximum(z, 0.0)
    t = jnp.dot(z, w2_ref[...], preferred_element_type=jnp.float32) + b2_ref[...]
    h = jnp.maximum(g_ref[...] * t + be_ref[...], 0.0)
    o_ref[...] = jnp.dot(h, w1n_ref[...], preferred_element_type=jnp.float32)


def _mlp_block_tc(y, a0, a1, em, b1, W2, b2, g, be, W1n):
    M, H = y.shape
    F = W1n.shape[1]
    grid = M // _ROWS
    row = lambda i: (i, 0)
    one = lambda i: (0, 0)
    return pl.pallas_call(
        _block_body,
        grid=(grid,),
        in_specs=[
            pl.BlockSpec((_ROWS, H), row),
            pl.BlockSpec((_ROWS, H), row),
            pl.BlockSpec((_ROWS, H), row),
            pl.BlockSpec((1, H), one),
            pl.BlockSpec((1, H), one),
            pl.BlockSpec((H, H), one),
            pl.BlockSpec((1, H), one),
            pl.BlockSpec((1, H), one),
            pl.BlockSpec((1, H), one),
            pl.BlockSpec((H, F), one),
        ],
        out_specs=pl.BlockSpec((_ROWS, F), row),
        out_shape=jax.ShapeDtypeStruct((M, F), jnp.float32),
    )(y, a0, a1, em, b1, W2, b2, g, be, W1n)


def _final_body(n_real, y_ref, a0_ref, a1_ref, em_ref, b1_ref, w2_ref,
                b2_ref, g_ref, be_ref, o_ref, acc_ref):
    i = pl.program_id(0)
    z = em_ref[...] * y_ref[...] + a0_ref[...] + a1_ref[...] + b1_ref[...]
    z = jnp.maximum(z, 0.0)
    rowid = lax.broadcasted_iota(jnp.int32, z.shape, 0) + i * _ROWS
    z = jnp.where(rowid < n_real, z, 0.0)
    part = jnp.sum(z, axis=0, keepdims=True)

    @pl.when(i == 0)
    def _():
        acc_ref[...] = jnp.zeros_like(acc_ref)

    acc_ref[...] += part

    @pl.when(i == pl.num_programs(0) - 1)
    def _():
        s = acc_ref[...] * (1.0 / n_real)
        t = jnp.dot(s, w2_ref[...], preferred_element_type=jnp.float32)
        o_ref[...] = g_ref[...] * (t + b2_ref[...]) + be_ref[...]


def _final_tc(y, a0, a1, em, b1, W2, b2, g, be, n_real):
    M, F = y.shape
    grid = M // _ROWS
    row = lambda i: (i, 0)
    one = lambda i: (0, 0)
    return pl.pallas_call(
        functools.partial(_final_body, n_real),
        grid=(grid,),
        in_specs=[
            pl.BlockSpec((_ROWS, F), row),
            pl.BlockSpec((_ROWS, F), row),
            pl.BlockSpec((_ROWS, F), row),
            pl.BlockSpec((1, F), one),
            pl.BlockSpec((1, F), one),
            pl.BlockSpec((F, F), one),
            pl.BlockSpec((1, F), one),
            pl.BlockSpec((1, F), one),
            pl.BlockSpec((1, F), one),
        ],
        out_specs=pl.BlockSpec((1, F), one),
        out_shape=jax.ShapeDtypeStruct((1, F), jnp.float32),
        scratch_shapes=[pltpu.VMEM((1, F), jnp.float32)],
    )(y, a0, a1, em, b1, W2, b2, g, be)


def _round_up(v, m):
    return (v + m - 1) // m * m


def kernel(x, edge_index, W1_0, b1_0, W2_0, b2_0, eps0, gamma0, beta0,
           W1_1, b1_1, W2_1, b2_1, eps1, gamma1, beta1):
    N, _ = x.shape
    H = W1_0.shape[1]
    OUT = W1_1.shape[1]
    E = edge_index.shape[1]
    NW = _NC * _NS

    n_pad = _round_up(N + 1, _ROWS)
    nb = -(-E // (NW * _BATCH))
    e_pad = NW * _BATCH * nb

    x_p = jnp.pad(x.astype(jnp.float32), ((0, n_pad - N), (0, 0)))
    src = edge_index[0]
    dst = edge_index[1]
    src_w = jnp.concatenate(
        [src, jnp.zeros((e_pad - E,), jnp.int32)]).reshape(NW, nb, _BATCH)
    dst_w = jnp.concatenate(
        [dst, jnp.full((e_pad - E,), N, jnp.int32)]).reshape(NW, nb, _BATCH)

    bn_scale = 1.0 / jnp.sqrt(1.0 + _BN_EPS)
    em0 = (1.0 + eps0) * jnp.ones((1, H), jnp.float32)
    em1 = (1.0 + eps1) * jnp.ones((1, OUT), jnp.float32)
    g0 = (gamma0 * bn_scale).reshape(1, H)
    g1 = (gamma1 * bn_scale).reshape(1, OUT)

    y0 = _linear_tc(x_p, W1_0)
    acc0 = _segment_sum_sc(y0, src_w, dst_w, n_pad, H, nb)
    y1 = _mlp_block_tc(y0, acc0[0], acc0[1], em0, b1_0.reshape(1, H),
                       W2_0, b2_0.reshape(1, H), g0, beta0.reshape(1, H),
                       W1_1)
    acc1 = _segment_sum_sc(y1, src_w, dst_w, n_pad, OUT, nb)
    return _final_tc(y1, acc1[0], acc1[1], em1, b1_1.reshape(1, OUT),
                     W2_1, b2_1.reshape(1, OUT), g1, beta1.reshape(1, OUT),
                     N)
